# baseline (speedup 1.0000x reference)
import functools
import jax
import jax.numpy as jnp
from jax import lax
from jax.experimental import pallas as pl
from jax.experimental.pallas import tpu as pltpu
from jax.experimental.pallas import tpu_sc as plsc

NUM_CORES = 2
NUM_SUBCORES = 16
NUM_WORKERS = NUM_CORES * NUM_SUBCORES
LANES = 16
CHUNK = 128
NBUF = 8
AHEAD = 4
PAD = 128


def _make_sc_kernel(B, D):
    assert B % (NUM_WORKERS * CHUNK * NBUF) == 0
    b_per_w = B // NUM_WORKERS
    n_chunks = b_per_w // CHUNK

    mesh = plsc.VectorSubcoreMesh(core_axis_name="c", subcore_axis_name="s")

    @functools.partial(
        pl.kernel,
        out_type=jax.ShapeDtypeStruct((B, PAD), jnp.float32),
        mesh=mesh,
        scratch_types=[
            pltpu.VMEM((n_chunks, CHUNK), jnp.int32),
            [pltpu.VMEM((CHUNK, D), jnp.float32) for _ in range(NBUF)],
            [pltpu.SemaphoreType.DMA for _ in range(NBUF)],
            [pltpu.SemaphoreType.DMA for _ in range(NBUF)],
        ],
        compiler_params=pltpu.CompilerParams(
            use_tc_tiling_on_sc=False, needs_layout_passes=False
        ),
    )
    def emb_kernel(tokens_hbm, table_hbm, out_hbm, idx_v, rows, gsems, wsems):
        wid = lax.axis_index("s") * NUM_CORES + lax.axis_index("c")
        base = wid * b_per_w
        pltpu.sync_copy(tokens_hbm.at[pl.ds(wid * n_chunks, n_chunks)], idx_v)

        for k in range(AHEAD):
            pltpu.async_copy(table_hbm.at[idx_v.at[k]], rows[k], gsems[k])

        def wcopy(b, k):
            return pltpu.make_async_copy(
                rows[b],
                out_hbm.at[pl.ds(base + k * CHUNK, CHUNK), pl.ds(0, D)],
                wsems[b],
            )

        def body(g, carry):
            for b in range(NBUF):
                k = g * NBUF + b
                pltpu.make_async_copy(
                    table_hbm.at[idx_v.at[k]], rows[b], gsems[b]
                ).wait()
                wcopy(b, k).start()
                nb = (b + AHEAD) % NBUF
                kn = k + AHEAD

                @pl.when(kn < n_chunks)
                def _():
                    @pl.when(kn >= NBUF)
                    def _():
                        wcopy(nb, kn - NBUF).wait()

                    pltpu.async_copy(
                        table_hbm.at[idx_v.at[kn]], rows[nb], gsems[nb]
                    )

            return carry

        lax.fori_loop(0, n_chunks // NBUF, body, 0)

        for b in range(NBUF):
            wcopy(b, n_chunks - NBUF + b).wait()

    return emb_kernel


@jax.jit
def kernel(tokens, emb_table):
    B = tokens.shape[0] * tokens.shape[1]
    V, D = emb_table.shape
    padded = jnp.pad(emb_table, ((0, 0), (0, PAD - D))) * float(D) ** 0.5
    view = padded.reshape(V * (PAD // D), D)
    flat = (tokens.reshape(B // CHUNK, CHUNK) * (PAD // D)).astype(jnp.int32)
    out_pad = _make_sc_kernel(B, D)(flat, view)
    out = out_pad[:, :D].reshape(tokens.shape + (D,))
    return out
